# baseline (speedup 1.0000x reference)
import functools

import jax
import jax.numpy as jnp
from jax import lax
from jax.experimental import pallas as pl
from jax.experimental.pallas import tpu as pltpu

N_NODES = 15
N_EDGES = 256
NP = 16
HID = 512


def _fused_body(ei_ref, eiT_ref, xT_ref, WgT_ref, bg_ref,
                WihT_r_ref, WihT_z_ref, WihT_n_ref,
                WhhT_ref, Whh_ref, Bhh_n_ref, Bhh_nc_ref,
                B_r_ref, B_z_ref, B_n_ref, wlin_ref, blin_ref, out_ref,
                gi_r_ref, gi_z_ref, gi_n_ref, hseq_ref):
    f32 = jnp.float32
    src_row = ei_ref[0:1, :]
    dst_row = ei_ref[1:2, :]
    src_col = eiT_ref[:, 0:1]
    dst_col = eiT_ref[:, 1:2]
    node_col = lax.broadcasted_iota(jnp.int32, (NP, 1), 0)
    node_row = lax.broadcasted_iota(jnp.int32, (1, NP), 1)
    ST = (src_row == node_col).astype(f32)
    DT = (dst_row == node_col).astype(f32)
    D = (dst_col == node_row).astype(f32)

    deg_col = jnp.sum(DT, axis=1, keepdims=True) + 1.0
    dinv_col = lax.rsqrt(deg_col)
    dinv_src = jnp.sum(ST * dinv_col, axis=0, keepdims=True)
    dinv_dst = jnp.sum(DT * dinv_col, axis=0, keepdims=True)
    norm_row = dinv_src * dinv_dst

    AT = jnp.dot(ST * norm_row, D, preferred_element_type=f32)
    eye = (node_col == node_row).astype(f32)
    AT = AT + eye * (1.0 / deg_col)

    hi = lax.Precision.HIGHEST
    aggT = jnp.dot(xT_ref[:, :], AT, preferred_element_type=f32, precision=hi)
    hT = jnp.dot(WgT_ref[:, :], aggT, preferred_element_type=f32, precision=hi)
    hT = jnp.maximum(hT + bg_ref[:, :], 0.0)

    gi_r_ref[:, :] = jnp.dot(hT, WihT_r_ref[:, :], preferred_element_type=f32,
                             precision=hi) + B_r_ref[:, :]
    gi_z_ref[:, :] = jnp.dot(hT, WihT_z_ref[:, :], preferred_element_type=f32,
                             precision=hi) + B_z_ref[:, :]
    gi_n_ref[:, :] = jnp.dot(hT, WihT_n_ref[:, :], preferred_element_type=f32,
                             precision=hi) + B_n_ref[:, :]

    WhT_r = WhhT_ref[:, 0:NP]
    WhT_z = WhhT_ref[:, NP:2 * NP]
    WhT_n = WhhT_ref[:, 2 * NP:3 * NP]
    Wh_r = Whh_ref[0:NP, :]
    Wh_z = Whh_ref[NP:2 * NP, :]
    Wh_n = Whh_ref[2 * NP:3 * NP, :]
    bhh_n = Bhh_n_ref[:, :]
    bhh_nc = Bhh_nc_ref[:, :]

    def step(t, carry):
        h_row, h_col = carry
        hcb = jnp.broadcast_to(h_col, (NP, NP))
        hrb = jnp.broadcast_to(h_row, (NP, NP))
        gh_r = jnp.sum(hcb * WhT_r, axis=0, keepdims=True)
        gh_z = jnp.sum(hcb * WhT_z, axis=0, keepdims=True)
        gh_n = jnp.sum(hcb * WhT_n, axis=0, keepdims=True)
        gc_r = jnp.sum(hrb * Wh_r, axis=1, keepdims=True)
        gc_z = jnp.sum(hrb * Wh_z, axis=1, keepdims=True)
        gc_n = jnp.sum(hrb * Wh_n, axis=1, keepdims=True)
        gi_r = gi_r_ref[pl.ds(t, 1), :]
        gi_z = gi_z_ref[pl.ds(t, 1), :]
        gi_n = gi_n_ref[pl.ds(t, 1), :]
        gi_rc = jnp.transpose(gi_r)
        gi_zc = jnp.transpose(gi_z)
        gi_nc = jnp.transpose(gi_n)
        r_row = jax.nn.sigmoid(gi_r + gh_r)
        z_row = jax.nn.sigmoid(gi_z + gh_z)
        ng_row = jnp.tanh(gi_n + r_row * (gh_n + bhh_n))
        r_col = jax.nn.sigmoid(gi_rc + gc_r)
        z_col = jax.nn.sigmoid(gi_zc + gc_z)
        ng_col = jnp.tanh(gi_nc + r_col * (gc_n + bhh_nc))
        h_row = ng_row + z_row * (h_row - ng_row)
        h_col = ng_col + z_col * (h_col - ng_col)
        hseq_ref[pl.ds(t, 1), :] = h_row
        return h_row, h_col

    h0 = (jnp.zeros((1, NP), dtype=f32), jnp.zeros((NP, 1), dtype=f32))
    lax.fori_loop(0, HID, step, h0, unroll=8)
    acc = jnp.dot(wlin_ref[:, :], hseq_ref[:, :], preferred_element_type=f32,
                  precision=hi)
    out_ref[:, :] = acc + blin_ref[0:1, 0:1]


def _pad2(a, r, c):
    return jnp.pad(a, ((0, r - a.shape[0]), (0, c - a.shape[1])))


@functools.partial(jax.jit, static_argnames=())
def kernel(x, edge_index, W_gcn, b_gcn, W_ih, W_hh, b_ih, b_hh, W_lin, b_lin):
    f32 = jnp.float32
    Hd = W_hh.shape[1]
    ei = edge_index.astype(jnp.int32)
    eiT = ei.T
    xT = _pad2(x.T.astype(f32), HID, NP)
    WgT = W_gcn.T.astype(f32)
    bg = b_gcn.reshape(HID, 1).astype(f32)

    def gate(W, i):
        return _pad2(W[i * Hd:(i + 1) * Hd, :].T.astype(f32), NP, NP)

    def brow(b):
        return _pad2(b.reshape(1, Hd).astype(f32), 1, NP)

    def gate_nt(W, i):
        return _pad2(W[i * Hd:(i + 1) * Hd, :].astype(f32), NP, NP)

    WhhT = jnp.concatenate([gate(W_hh, 0), gate(W_hh, 1), gate(W_hh, 2)], axis=1)
    Whh = jnp.concatenate([gate_nt(W_hh, 0), gate_nt(W_hh, 1), gate_nt(W_hh, 2)],
                          axis=0)
    B_r = brow(b_ih[0:Hd] + b_hh[0:Hd])
    B_z = brow(b_ih[Hd:2 * Hd] + b_hh[Hd:2 * Hd])
    B_n = brow(b_ih[2 * Hd:])
    Bhh_n = brow(b_hh[2 * Hd:])
    Bhh_nc = Bhh_n.reshape(NP, 1)
    wlin = W_lin.astype(f32)
    blin = b_lin.reshape(1, 1).astype(f32)

    acc = pl.pallas_call(
        _fused_body,
        out_shape=jax.ShapeDtypeStruct((1, NP), f32),
        scratch_shapes=[pltpu.VMEM((HID, NP), f32)] * 4,
    )(ei, eiT, xT, WgT, bg, gate(W_ih, 0), gate(W_ih, 1), gate(W_ih, 2),
      WhhT, Whh, Bhh_n, Bhh_nc, B_r, B_z, B_n, wlin, blin)

    return acc[0, :N_NODES].reshape(N_NODES, 1)


# speedup vs baseline: 1.0489x; 1.0489x over previous
import functools

import jax
import jax.numpy as jnp
from jax import lax
from jax.experimental import pallas as pl
from jax.experimental.pallas import tpu as pltpu

N_NODES = 15
N_EDGES = 256
NP = 16
HID = 512


def _padv(v, r, c):
    f32 = v.dtype
    if v.shape[1] < c:
        v = jnp.concatenate([v, jnp.zeros((v.shape[0], c - v.shape[1]), f32)],
                            axis=1)
    if v.shape[0] < r:
        v = jnp.concatenate([v, jnp.zeros((r - v.shape[0], v.shape[1]), f32)],
                            axis=0)
    return v


def _fused_body(ei_ref, x_ref, Wg_ref, bg_ref, wih_ref, whh_ref,
                bih_ref, bhh_ref, wlin_ref, blin_ref, out_ref,
                gi_r_ref, gi_z_ref, gi_n_ref, hseq_ref):
    f32 = jnp.float32
    hi = lax.Precision.HIGHEST
    Hd = N_NODES

    src_row = ei_ref[0:1, :]
    dst_row = ei_ref[1:2, :]
    node_col = lax.broadcasted_iota(jnp.int32, (NP, 1), 0)
    node_row = lax.broadcasted_iota(jnp.int32, (1, NP), 1)
    ST = (src_row == node_col).astype(f32)
    DT = (dst_row == node_col).astype(f32)

    deg_col = jnp.sum(DT, axis=1, keepdims=True) + 1.0
    dinv_col = lax.rsqrt(deg_col)
    dinv_src = jnp.sum(ST * dinv_col, axis=0, keepdims=True)
    dinv_dst = jnp.sum(DT * dinv_col, axis=0, keepdims=True)
    norm_row = dinv_src * dinv_dst

    AT = lax.dot_general(ST * norm_row, DT, (((1,), (1,)), ((), ())),
                         preferred_element_type=f32)
    eye = (node_col == node_row).astype(f32)
    AT = AT + eye * (1.0 / deg_col)

    agg = lax.dot_general(AT[0:Hd, :], x_ref[:, :], (((0,), (0,)), ((), ())),
                          preferred_element_type=f32, precision=hi)
    hT = lax.dot_general(Wg_ref[:, :], agg, (((0,), (1,)), ((), ())),
                         preferred_element_type=f32, precision=hi)
    hT = jnp.maximum(hT + bg_ref[:, :], 0.0)
    hT15 = hT[:, 0:Hd]

    def gi(g, bias):
        W = _padv(wih_ref[g * Hd:(g + 1) * Hd, :], NP, Hd)
        GI = lax.dot_general(hT15, W, (((1,), (1,)), ((), ())),
                             preferred_element_type=f32, precision=hi)
        return GI + _padv(bias, 1, NP)

    gi_r_ref[:, :] = gi(0, bih_ref[0:1, 0:Hd] + bhh_ref[0:1, 0:Hd])
    gi_z_ref[:, :] = gi(1, bih_ref[0:1, Hd:2 * Hd] + bhh_ref[0:1, Hd:2 * Hd])
    gi_n_ref[:, :] = gi(2, bih_ref[0:1, 2 * Hd:3 * Hd])

    def whT(g):
        return _padv(jnp.transpose(whh_ref[g * Hd:(g + 1) * Hd, :]), NP, NP)

    WhT_r, WhT_z, WhT_n = whT(0), whT(1), whT(2)
    bhh_n = _padv(bhh_ref[0:1, 2 * Hd:3 * Hd], 1, NP)

    def step(t, carry):
        h_row, h_col = carry
        hcb = jnp.broadcast_to(h_col, (NP, NP))
        gh_r = jnp.sum(hcb * WhT_r, axis=0, keepdims=True)
        gh_z = jnp.sum(hcb * WhT_z, axis=0, keepdims=True)
        gh_n = jnp.sum(hcb * WhT_n, axis=0, keepdims=True)
        r = jax.nn.sigmoid(gi_r_ref[pl.ds(t, 1), :] + gh_r)
        z = jax.nn.sigmoid(gi_z_ref[pl.ds(t, 1), :] + gh_z)
        ng = jnp.tanh(gi_n_ref[pl.ds(t, 1), :] + r * (gh_n + bhh_n))
        h_row = ng + z * (h_row - ng)
        z_col = jnp.transpose(z)
        ng_col = jnp.transpose(ng)
        h_col = ng_col + z_col * (h_col - ng_col)
        hseq_ref[pl.ds(t, 1), :] = h_row
        return h_row, h_col

    h0 = (jnp.zeros((1, NP), f32), jnp.zeros((NP, 1), f32))
    lax.fori_loop(0, HID, step, h0, unroll=8)

    acc = jnp.dot(wlin_ref[:, :], hseq_ref[:, :], preferred_element_type=f32,
                  precision=hi)
    acc = acc + blin_ref[0:1, 0:1]
    out_ref[:, :] = jnp.transpose(acc)[0:Hd, :]


@jax.jit
def kernel(x, edge_index, W_gcn, b_gcn, W_ih, W_hh, b_ih, b_hh, W_lin, b_lin):
    f32 = jnp.float32
    return pl.pallas_call(
        _fused_body,
        out_shape=jax.ShapeDtypeStruct((N_NODES, 1), f32),
        scratch_shapes=[pltpu.VMEM((HID, NP), f32)] * 4,
    )(edge_index.astype(jnp.int32), x.astype(f32), W_gcn.astype(f32),
      b_gcn.reshape(HID, 1).astype(f32), W_ih.astype(f32), W_hh.astype(f32),
      b_ih.reshape(1, 45).astype(f32), b_hh.reshape(1, 45).astype(f32),
      W_lin.astype(f32), b_lin.reshape(1, 1).astype(f32))
